# baseline (speedup 1.0000x reference)
import functools

import jax
import jax.numpy as jnp
from jax import lax
from jax.experimental import pallas as pl
from jax.experimental.pallas import tpu as pltpu
from jax.experimental.pallas import tpu_sc as plsc


D = 1024
B = 4
_SB = 512
_CH = 8
_NW = 32
_LANES = 16


def _add_kernel(x_ref, pe_ref, o_ref):
    o_ref[...] = x_ref[...] + pe_ref[...][None, :, :]


def _tc_kernel(x, pe_table):
    batch, seq_len, d = x.shape
    nb = seq_len // _SB
    return pl.pallas_call(
        _add_kernel,
        grid=(nb,),
        in_specs=[
            pl.BlockSpec((batch, _SB, d), lambda i: (0, i, 0)),
            pl.BlockSpec((_SB, d), lambda i: (i, 0)),
        ],
        out_specs=pl.BlockSpec((batch, _SB, d), lambda i: (0, i, 0)),
        out_shape=jax.ShapeDtypeStruct((batch, seq_len, d), x.dtype),
    )(x, pe_table)


def _make_sc_add(seq_len):
    seq_per_w = seq_len // _NW
    n_chunks = seq_per_w // _CH
    cols = D // _LANES
    mesh = plsc.VectorSubcoreMesh(core_axis_name="c", subcore_axis_name="s")

    @functools.partial(
        pl.kernel,
        out_type=jax.ShapeDtypeStruct((B, seq_len, D), jnp.float32),
        mesh=mesh,
        scratch_types=[
            pltpu.VMEM((B, _CH, D), jnp.float32),
            pltpu.VMEM((B, _CH, D), jnp.float32),
            pltpu.VMEM((B, _CH, D), jnp.float32),
            pltpu.VMEM((_CH, D), jnp.float32),
            pltpu.VMEM((_CH, D), jnp.float32),
            pltpu.VMEM((_CH, D), jnp.float32),
            pltpu.SemaphoreType.DMA,
            pltpu.SemaphoreType.DMA,
            pltpu.SemaphoreType.DMA,
            pltpu.SemaphoreType.DMA,
            pltpu.SemaphoreType.DMA,
            pltpu.SemaphoreType.DMA,
        ],
    )
    def sc_add(x_hbm, pe_hbm, out_hbm,
               xb0, xb1, xb2, pb0, pb1, pb2, ls0, ls1, ls2, os0, os1, os2):
        nc = 2
        wid = lax.axis_index("s") * nc + lax.axis_index("c")
        seq_base = wid * seq_per_w

        def load_copies(c, xb, pb, ls):
            row = seq_base + c * _CH
            cps = [
                pltpu.make_async_copy(
                    x_hbm.at[b, pl.ds(row, _CH)], xb.at[b], ls
                )
                for b in range(B)
            ]
            cps.append(pltpu.make_async_copy(pe_hbm.at[pl.ds(row, _CH)], pb, ls))
            return cps

        def out_copies(c, xb, os):
            row = seq_base + c * _CH
            return [
                pltpu.make_async_copy(
                    xb.at[b], out_hbm.at[b, pl.ds(row, _CH)], os
                )
                for b in range(B)
            ]

        def start(cps):
            for cp in cps:
                cp.start()

        def drain(cps):
            for cp in cps:
                cp.wait()

        def compute(xb, pb):
            def col_body(cc, _):
                sl = pl.ds(cc * _LANES, _LANES)
                for r in range(_CH):
                    pv = pb[r, sl]
                    for b in range(B):
                        plsc.addupdate(xb.at[b, r, sl], pv)
                return 0

            lax.fori_loop(0, cols, col_body, 0, unroll=2)

        bufs = ((xb0, pb0, ls0, os0), (xb1, pb1, ls1, os1), (xb2, pb2, ls2, os2))

        def chunk_step(c, j):
            xb, pb, ls, os = bufs[j]
            xbn, pbn, lsn, osn = bufs[(j + 2) % 3]
            drain(load_copies(c, xb, pb, ls))

            @pl.when(c > 0)
            def _():
                drain(out_copies(c - 1, xbn, osn))

            @pl.when(c + 2 < n_chunks)
            def _():
                start(load_copies(c + 2, xbn, pbn, lsn))

            compute(xb, pb)
            start(out_copies(c, xb, os))

        start(load_copies(0, xb0, pb0, ls0))
        start(load_copies(1, xb1, pb1, ls1))

        def triple_body(k, _):
            c = 3 * k
            chunk_step(c, 0)
            chunk_step(c + 1, 1)
            chunk_step(c + 2, 2)
            return 0

        n_triples = n_chunks // 3
        lax.fori_loop(0, n_triples, triple_body, 0)
        chunk_step(n_chunks - 2, 0)
        chunk_step(n_chunks - 1, 1)
        drain(out_copies(n_chunks - 1, xb1, os1))

    return sc_add


def _sc_kernel(x, pe_table):
    fn = _make_sc_add(x.shape[1])
    return fn(x, pe_table)


def kernel(x, pe_table):
    return _sc_kernel(x, pe_table)


# speedup vs baseline: 1.2200x; 1.2200x over previous
import functools

import jax
import jax.numpy as jnp
from jax import lax
from jax.experimental import pallas as pl
from jax.experimental.pallas import tpu as pltpu
from jax.experimental.pallas import tpu_sc as plsc


D = 1024
B = 4
_SB = 512
_CH = 8
_NW = 32
_LANES = 16


def _add_kernel(x_ref, pe_ref, o_ref):
    o_ref[...] = x_ref[...] + pe_ref[...][None, :, :]


def _tc_kernel(x, pe_table):
    batch, seq_len, d = x.shape
    nb = seq_len // _SB
    return pl.pallas_call(
        _add_kernel,
        grid=(nb,),
        in_specs=[
            pl.BlockSpec((batch, _SB, d), lambda i: (0, i, 0)),
            pl.BlockSpec((_SB, d), lambda i: (i, 0)),
        ],
        out_specs=pl.BlockSpec((batch, _SB, d), lambda i: (0, i, 0)),
        out_shape=jax.ShapeDtypeStruct((batch, seq_len, d), x.dtype),
    )(x, pe_table)


def _make_sc_add(seq_len):
    seq_per_w = seq_len // _NW
    n_chunks = seq_per_w // _CH
    cols = D // _LANES
    mesh = plsc.VectorSubcoreMesh(core_axis_name="c", subcore_axis_name="s")

    @functools.partial(
        pl.kernel,
        out_type=jax.ShapeDtypeStruct((B, seq_len, D), jnp.float32),
        mesh=mesh,
        scratch_types=[
            pltpu.VMEM((B, _CH, D), jnp.float32),
            pltpu.VMEM((B, _CH, D), jnp.float32),
            pltpu.VMEM((B, _CH, D), jnp.float32),
            pltpu.VMEM((_CH, D), jnp.float32),
            pltpu.VMEM((_CH, D), jnp.float32),
            pltpu.VMEM((_CH, D), jnp.float32),
            pltpu.SemaphoreType.DMA,
            pltpu.SemaphoreType.DMA,
            pltpu.SemaphoreType.DMA,
            pltpu.SemaphoreType.DMA,
            pltpu.SemaphoreType.DMA,
            pltpu.SemaphoreType.DMA,
        ],
    )
    def sc_add(x_hbm, pe_hbm, out_hbm,
               xb0, xb1, xb2, pb0, pb1, pb2, ls0, ls1, ls2, os0, os1, os2):
        nc = 2
        wid = lax.axis_index("s") * nc + lax.axis_index("c")
        seq_base = wid * seq_per_w

        def load_copies(c, xb, pb, ls):
            row = seq_base + c * _CH
            cps = [
                pltpu.make_async_copy(
                    x_hbm.at[b, pl.ds(row, _CH)], xb.at[b], ls
                )
                for b in range(B)
            ]
            cps.append(pltpu.make_async_copy(pe_hbm.at[pl.ds(row, _CH)], pb, ls))
            return cps

        def out_copies(c, xb, os):
            row = seq_base + c * _CH
            return [
                pltpu.make_async_copy(
                    xb.at[b], out_hbm.at[b, pl.ds(row, _CH)], os
                )
                for b in range(B)
            ]

        def start(cps):
            for cp in cps:
                cp.start()

        def drain(cps):
            for cp in cps:
                cp.wait()

        def compute(xb, pb):
            def row_body(r, _):
                def col_body(cc, _):
                    sl = pl.ds(cc * _LANES, _LANES)
                    pv = pb[r, sl]
                    for b in range(B):
                        plsc.addupdate(xb.at[b, r, sl], pv)
                    return 0

                lax.fori_loop(0, cols, col_body, 0, unroll=8)
                return 0

            lax.fori_loop(0, _CH, row_body, 0)

        bufs = ((xb0, pb0, ls0, os0), (xb1, pb1, ls1, os1), (xb2, pb2, ls2, os2))

        def chunk_step(c, j):
            xb, pb, ls, os = bufs[j]
            xbn, pbn, lsn, osn = bufs[(j + 2) % 3]
            drain(load_copies(c, xb, pb, ls))

            @pl.when(c > 0)
            def _():
                drain(out_copies(c - 1, xbn, osn))

            @pl.when(c + 2 < n_chunks)
            def _():
                start(load_copies(c + 2, xbn, pbn, lsn))

            compute(xb, pb)
            start(out_copies(c, xb, os))

        start(load_copies(0, xb0, pb0, ls0))
        start(load_copies(1, xb1, pb1, ls1))

        def triple_body(k, _):
            c = 3 * k
            chunk_step(c, 0)
            chunk_step(c + 1, 1)
            chunk_step(c + 2, 2)
            return 0

        n_triples = n_chunks // 3
        lax.fori_loop(0, n_triples, triple_body, 0)
        chunk_step(n_chunks - 2, 0)
        chunk_step(n_chunks - 1, 1)
        drain(out_copies(n_chunks - 1, xb1, os1))

    return sc_add


def _sc_kernel(x, pe_table):
    fn = _make_sc_add(x.shape[1])
    return fn(x, pe_table)


def kernel(x, pe_table):
    return _sc_kernel(x, pe_table)


# speedup vs baseline: 1.2218x; 1.0015x over previous
import functools

import jax
import jax.numpy as jnp
from jax import lax
from jax.experimental import pallas as pl
from jax.experimental.pallas import tpu as pltpu
from jax.experimental.pallas import tpu_sc as plsc


D = 1024
B = 4
_SB = 512
_CH = 8
_NW = 32
_LANES = 16


def _add_kernel(x_ref, pe_ref, o_ref):
    o_ref[...] = x_ref[...] + pe_ref[...][None, :, :]


def _tc_kernel(x, pe_table):
    batch, seq_len, d = x.shape
    nb = seq_len // _SB
    return pl.pallas_call(
        _add_kernel,
        grid=(nb,),
        in_specs=[
            pl.BlockSpec((batch, _SB, d), lambda i: (0, i, 0)),
            pl.BlockSpec((_SB, d), lambda i: (i, 0)),
        ],
        out_specs=pl.BlockSpec((batch, _SB, d), lambda i: (0, i, 0)),
        out_shape=jax.ShapeDtypeStruct((batch, seq_len, d), x.dtype),
    )(x, pe_table)


def _make_sc_add(seq_len):
    seq_per_w = seq_len // _NW
    n_chunks = seq_per_w // _CH
    cols = D // _LANES
    mesh = plsc.VectorSubcoreMesh(core_axis_name="c", subcore_axis_name="s")

    @functools.partial(
        pl.kernel,
        out_type=jax.ShapeDtypeStruct((B, seq_len, D), jnp.float32),
        mesh=mesh,
        scratch_types=[
            pltpu.VMEM((B, _CH, D), jnp.float32),
            pltpu.VMEM((B, _CH, D), jnp.float32),
            pltpu.VMEM((B, _CH, D), jnp.float32),
            pltpu.VMEM((_CH, D), jnp.float32),
            pltpu.VMEM((_CH, D), jnp.float32),
            pltpu.VMEM((_CH, D), jnp.float32),
            pltpu.SemaphoreType.DMA,
            pltpu.SemaphoreType.DMA,
            pltpu.SemaphoreType.DMA,
            pltpu.SemaphoreType.DMA,
            pltpu.SemaphoreType.DMA,
            pltpu.SemaphoreType.DMA,
        ],
    )
    def sc_add(x_hbm, pe_hbm, out_hbm,
               xb0, xb1, xb2, pb0, pb1, pb2, ls0, ls1, ls2, os0, os1, os2):
        nc = 2
        wid = lax.axis_index("s") * nc + lax.axis_index("c")
        seq_base = wid * seq_per_w

        def load_copies(c, xb, pb, ls):
            row = seq_base + c * _CH
            cps = [
                pltpu.make_async_copy(
                    x_hbm.at[b, pl.ds(row, _CH)], xb.at[b], ls
                )
                for b in range(B)
            ]
            cps.append(pltpu.make_async_copy(pe_hbm.at[pl.ds(row, _CH)], pb, ls))
            return cps

        def out_copies(c, xb, os):
            row = seq_base + c * _CH
            return [
                pltpu.make_async_copy(
                    xb.at[b], out_hbm.at[b, pl.ds(row, _CH)], os
                )
                for b in range(B)
            ]

        def start(cps):
            for cp in cps:
                cp.start()

        def drain(cps):
            for cp in cps:
                cp.wait()

        def compute(xb, pb):
            def row_body(r, _):
                @plsc.parallel_loop(0, cols, 1, unroll=8)
                def col_body(cc):
                    sl = pl.ds(cc * _LANES, _LANES)
                    pv = pb[r, sl]
                    for b in range(B):
                        plsc.addupdate(xb.at[b, r, sl], pv)

                return 0

            lax.fori_loop(0, _CH, row_body, 0)

        bufs = ((xb0, pb0, ls0, os0), (xb1, pb1, ls1, os1), (xb2, pb2, ls2, os2))

        def chunk_step(c, j):
            xb, pb, ls, os = bufs[j]
            xbn, pbn, lsn, osn = bufs[(j + 2) % 3]
            drain(load_copies(c, xb, pb, ls))

            @pl.when(c > 0)
            def _():
                drain(out_copies(c - 1, xbn, osn))

            @pl.when(c + 2 < n_chunks)
            def _():
                start(load_copies(c + 2, xbn, pbn, lsn))

            compute(xb, pb)
            start(out_copies(c, xb, os))

        start(load_copies(0, xb0, pb0, ls0))
        start(load_copies(1, xb1, pb1, ls1))

        def triple_body(k, _):
            c = 3 * k
            chunk_step(c, 0)
            chunk_step(c + 1, 1)
            chunk_step(c + 2, 2)
            return 0

        n_triples = n_chunks // 3
        lax.fori_loop(0, n_triples, triple_body, 0)
        chunk_step(n_chunks - 2, 0)
        chunk_step(n_chunks - 1, 1)
        drain(out_copies(n_chunks - 1, xb1, os1))

    return sc_add


def _sc_kernel(x, pe_table):
    fn = _make_sc_add(x.shape[1])
    return fn(x, pe_table)


def kernel(x, pe_table):
    return _sc_kernel(x, pe_table)
